# Initial kernel scaffold; baseline (speedup 1.0000x reference)
import functools

import jax
import jax.numpy as jnp
from jax import lax
from jax.experimental import pallas as pl
from jax.experimental.pallas import tpu as pltpu
from jax.experimental.pallas import tpu_sc as plsc

N = 10000
E = 320000
D = 128
NC = 2
NS = 16
L = 16
CK = 125
DEG_CH = (E // NS) // CK
SC_CH = (E // (NC * NS)) // CK
RPT = N // NS

_MESH = plsc.VectorSubcoreMesh(
    core_axis_name="c", subcore_axis_name="s", num_cores=NC, num_subcores=NS
)


def _degree_body(idx_hbm, out_hbm, idx_v, ones_v, zb_v, acc):
    c = lax.axis_index("c")
    s = lax.axis_index("s")
    ones16 = jnp.full((L,), 1.0, jnp.float32)
    zeros16 = jnp.zeros((L,), jnp.float32)

    @pl.loop(0, CK)
    def _(j):
        ones_v[j] = ones16

    @pl.loop(0, RPT)
    def _(j):
        zb_v[j] = zeros16

    base = s * RPT
    pltpu.sync_copy(zb_v, acc.at[pl.ds(base, RPT)])
    plsc.subcore_barrier()

    pltpu.sync_copy(idx_hbm.at[c, s], idx_v)

    @pl.loop(0, DEG_CH)
    def _(j):
        pltpu.sync_copy(ones_v, acc.at[idx_v.at[j]], add=True)

    plsc.subcore_barrier()
    pltpu.sync_copy(acc.at[pl.ds(base, RPT)], out_hbm.at[c, pl.ds(base, RPT)])


_degree_call = functools.partial(
    pl.kernel,
    out_type=jax.ShapeDtypeStruct((NC, N, L), jnp.float32),
    mesh=_MESH,
    scratch_types=[
        pltpu.VMEM((DEG_CH, CK), jnp.int32),
        pltpu.VMEM((CK, L), jnp.float32),
        pltpu.VMEM((RPT, L), jnp.float32),
        pltpu.VMEM_SHARED((N, L), jnp.float32),
    ],
)(_degree_body)


def _scatter_body(h_hbm, sidx_hbm, ridx_hbm, out_hbm, sidx_v, ridx_v, rows_v, acc, sem):
    c = lax.axis_index("c")
    s = lax.axis_index("s")
    zeros16 = jnp.zeros((L,), jnp.float32)

    @pl.loop(0, CK)
    def _(j):
        for k in range(D // L):
            rows_v[j, pl.ds(k * L, L)] = zeros16

    base = s * RPT
    for k in range(RPT // CK):
        pltpu.sync_copy(rows_v, acc.at[pl.ds(base + k * CK, CK)])
    plsc.subcore_barrier()

    pltpu.sync_copy(sidx_hbm.at[c, s], sidx_v)
    pltpu.sync_copy(ridx_hbm.at[c, s], ridx_v)

    @pl.loop(0, SC_CH)
    def _(j):
        pltpu.async_copy(h_hbm.at[sidx_v.at[j]], rows_v, sem).wait()
        pltpu.sync_copy(rows_v, acc.at[ridx_v.at[j]], add=True)

    plsc.subcore_barrier()
    pltpu.sync_copy(acc.at[pl.ds(base, RPT)], out_hbm.at[c, pl.ds(base, RPT)])


_scatter_call = functools.partial(
    pl.kernel,
    out_type=jax.ShapeDtypeStruct((NC, N, D), jnp.float32),
    mesh=_MESH,
    scratch_types=[
        pltpu.VMEM((SC_CH, CK), jnp.int32),
        pltpu.VMEM((SC_CH, CK), jnp.int32),
        pltpu.VMEM((CK, D), jnp.float32),
        pltpu.VMEM_SHARED((N, D), jnp.float32),
        pltpu.SemaphoreType.DMA,
    ],
)(_scatter_body)


_BLK = 1000


def _l1_body(x_ref, w_ref, b_ref, d_ref, o_ref):
    h = jnp.tanh(
        jnp.dot(x_ref[...], w_ref[...], preferred_element_type=jnp.float32)
        + b_ref[...]
    )
    dcol = d_ref[...][:, 0:1]
    o_ref[...] = h * lax.rsqrt(jnp.maximum(dcol + 1.0, 1.0))


def _l2_body(p0_ref, p1_ref, h_ref, dr_ref, w_ref, b_ref, ds_ref, o_ref):
    agg = (p0_ref[...] + p1_ref[...] + h_ref[...]) * lax.rsqrt(
        jnp.maximum(dr_ref[...][:, 0:1] + 1.0, 1.0)
    )
    h2 = jnp.dot(agg, w_ref[...], preferred_element_type=jnp.float32) + b_ref[...]
    o_ref[...] = h2 * lax.rsqrt(jnp.maximum(ds_ref[...][:, 0:1], 1.0))


def _fin_body(q0_ref, q1_ref, dr_ref, o_ref):
    o_ref[...] = (q0_ref[...] + q1_ref[...]) * lax.rsqrt(
        jnp.maximum(dr_ref[...][:, 0:1], 1.0)
    )


def _row_spec(i_dim):
    return pl.BlockSpec((_BLK, i_dim), lambda i: (i, 0))


def _full_spec(r, c):
    return pl.BlockSpec((r, c), lambda i: (0, 0))


def _tc_layer1(x, W1, b1, degs):
    return pl.pallas_call(
        _l1_body,
        grid=(N // _BLK,),
        in_specs=[_row_spec(D), _full_spec(D, D), _full_spec(1, D), _row_spec(L)],
        out_specs=_row_spec(D),
        out_shape=jax.ShapeDtypeStruct((N, D), jnp.float32),
    )(x, W1, b1.reshape(1, D), degs)


def _tc_layer2(p0, p1, h1s, degr, W2, b2, degs):
    return pl.pallas_call(
        _l2_body,
        grid=(N // _BLK,),
        in_specs=[
            _row_spec(D),
            _row_spec(D),
            _row_spec(D),
            _row_spec(L),
            _full_spec(D, D),
            _full_spec(1, D),
            _row_spec(L),
        ],
        out_specs=_row_spec(D),
        out_shape=jax.ShapeDtypeStruct((N, D), jnp.float32),
    )(p0, p1, h1s, degr, W2, b2.reshape(1, D), degs)


def _tc_final(q0, q1, degr):
    return pl.pallas_call(
        _fin_body,
        grid=(N // _BLK,),
        in_specs=[_row_spec(D), _row_spec(D), _row_spec(L)],
        out_specs=_row_spec(D),
        out_shape=jax.ShapeDtypeStruct((N, D), jnp.float32),
    )(q0, q1, degr)


@jax.jit
def kernel(x, senders, receivers, W1, b1, W2, b2):
    senders = senders.astype(jnp.int32)
    receivers = receivers.astype(jnp.int32)

    deg_in = jnp.stack(
        [senders.reshape(NS, DEG_CH, CK), receivers.reshape(NS, DEG_CH, CK)]
    )
    deg = _degree_call(deg_in)
    degs, degr = deg[0], deg[1]

    h1s = _tc_layer1(x, W1, b1, degs)

    sidx = senders.reshape(NC, NS, SC_CH, CK)
    ridx = receivers.reshape(NC, NS, SC_CH, CK)
    p = _scatter_call(h1s, sidx, ridx)
    h2s = _tc_layer2(p[0], p[1], h1s, degr, W2, b2, degs)
    q = _scatter_call(h2s, sidx, ridx)
    return _tc_final(q[0], q[1], degr)


# speedup vs baseline: 6.8484x; 6.8484x over previous
import functools

import jax
import jax.numpy as jnp
from jax import lax
from jax.experimental import pallas as pl
from jax.experimental.pallas import tpu as pltpu
from jax.experimental.pallas import tpu_sc as plsc

N = 10000
E = 320000
D = 128
DH = D // 2
NC = 2
NS = 16
L = 16
CK = 125
NCH = (E // NS) // CK
STRIPE = 624
LAST_STRIPE = N - (NS - 1) * STRIPE

_MESH = plsc.VectorSubcoreMesh(
    core_axis_name="c", subcore_axis_name="s", num_cores=NC, num_subcores=NS
)
_SC_PARAMS = pltpu.CompilerParams(use_tc_tiling_on_sc=False)


def _stripe_out(acc, out_ref, s):
    base = s * STRIPE

    @pl.when(s < NS - 1)
    def _():
        pltpu.sync_copy(acc.at[pl.ds(base, STRIPE)], out_ref.at[pl.ds(base, STRIPE)])

    @pl.when(s == NS - 1)
    def _():
        pltpu.sync_copy(
            acc.at[pl.ds(base, LAST_STRIPE)], out_ref.at[pl.ds(base, LAST_STRIPE)]
        )


def _degree_body(idx_hbm, out_hbm, idx_v, ones_v, zb_v, acc):
    c = lax.axis_index("c")
    s = lax.axis_index("s")
    ones16 = jnp.full((L,), 1.0, jnp.float32)
    zeros16 = jnp.zeros((L,), jnp.float32)

    @pl.loop(0, CK)
    def _(j):
        ones_v[j] = ones16

    @pl.loop(0, LAST_STRIPE)
    def _(j):
        zb_v[j] = zeros16

    base = s * STRIPE
    pltpu.sync_copy(zb_v, acc.at[pl.ds(base, LAST_STRIPE)])
    plsc.subcore_barrier()

    pltpu.sync_copy(idx_hbm.at[c, s], idx_v)

    @pl.loop(0, NCH)
    def _(j):
        pltpu.sync_copy(ones_v, acc.at[idx_v.at[j]], add=True)

    plsc.subcore_barrier()
    _stripe_out(acc, out_hbm.at[c], s)


_degree_call = functools.partial(
    pl.kernel,
    out_type=jax.ShapeDtypeStruct((NC, N, L), jnp.float32),
    mesh=_MESH,
    scratch_types=[
        pltpu.VMEM((NCH, CK), jnp.int32),
        pltpu.VMEM((CK, L), jnp.float32),
        pltpu.VMEM((LAST_STRIPE, L), jnp.float32),
        pltpu.VMEM_SHARED((N, L), jnp.float32),
    ],
    compiler_params=_SC_PARAMS,
)(_degree_body)


def _scatter_body(
    hlo_hbm, hhi_hbm, sidx_hbm, ridx_hbm, olo_hbm, ohi_hbm,
    sidx_v, ridx_v, rows_v, zb_v, acc, sem,
):
    c = lax.axis_index("c")
    s = lax.axis_index("s")
    zeros16 = jnp.zeros((L,), jnp.float32)

    @pl.loop(0, LAST_STRIPE // 2)
    def _(j):
        for k in range(DH // L):
            zb_v[j, pl.ds(k * L, L)] = zeros16

    base = s * STRIPE
    pltpu.sync_copy(zb_v, acc.at[pl.ds(base, LAST_STRIPE // 2)])
    pltpu.sync_copy(zb_v, acc.at[pl.ds(base + LAST_STRIPE // 2, LAST_STRIPE // 2)])
    plsc.subcore_barrier()

    pltpu.sync_copy(sidx_hbm.at[s], sidx_v)
    pltpu.sync_copy(ridx_hbm.at[s], ridx_v)

    @pl.loop(0, NCH)
    def _(j):
        @pl.when(c == 0)
        def _():
            pltpu.async_copy(hlo_hbm.at[sidx_v.at[j]], rows_v, sem).wait()

        @pl.when(c == 1)
        def _():
            pltpu.async_copy(hhi_hbm.at[sidx_v.at[j]], rows_v, sem).wait()

        pltpu.sync_copy(rows_v, acc.at[ridx_v.at[j]], add=True)

    plsc.subcore_barrier()

    @pl.when(c == 0)
    def _():
        _stripe_out(acc, olo_hbm, s)

    @pl.when(c == 1)
    def _():
        _stripe_out(acc, ohi_hbm, s)


_scatter_call = functools.partial(
    pl.kernel,
    out_type=[
        jax.ShapeDtypeStruct((N, DH), jnp.float32),
        jax.ShapeDtypeStruct((N, DH), jnp.float32),
    ],
    mesh=_MESH,
    scratch_types=[
        pltpu.VMEM((NCH, CK), jnp.int32),
        pltpu.VMEM((NCH, CK), jnp.int32),
        pltpu.VMEM((CK, DH), jnp.float32),
        pltpu.VMEM((LAST_STRIPE // 2, DH), jnp.float32),
        pltpu.VMEM_SHARED((N, DH), jnp.float32),
        pltpu.SemaphoreType.DMA,
    ],
    compiler_params=_SC_PARAMS,
)(_scatter_body)


_BLK = 1000


def _l1_body(x_ref, w_ref, b_ref, d_ref, olo_ref, ohi_ref):
    h = jnp.tanh(
        jnp.dot(x_ref[...], w_ref[...], preferred_element_type=jnp.float32)
        + b_ref[...]
    )
    dcol = d_ref[...][:, 0:1]
    h = h * lax.rsqrt(jnp.maximum(dcol + 1.0, 1.0))
    olo_ref[...] = h[:, :DH]
    ohi_ref[...] = h[:, DH:]


def _l2_body(plo_ref, phi_ref, hlo_ref, hhi_ref, dr_ref, w_ref, b_ref, ds_ref,
             olo_ref, ohi_ref):
    agg = jnp.concatenate(
        [plo_ref[...] + hlo_ref[...], phi_ref[...] + hhi_ref[...]], axis=1
    ) * lax.rsqrt(jnp.maximum(dr_ref[...][:, 0:1] + 1.0, 1.0))
    h2 = jnp.dot(agg, w_ref[...], preferred_element_type=jnp.float32) + b_ref[...]
    h2 = h2 * lax.rsqrt(jnp.maximum(ds_ref[...][:, 0:1], 1.0))
    olo_ref[...] = h2[:, :DH]
    ohi_ref[...] = h2[:, DH:]


def _fin_body(qlo_ref, qhi_ref, dr_ref, o_ref):
    o_ref[...] = jnp.concatenate(
        [qlo_ref[...], qhi_ref[...]], axis=1
    ) * lax.rsqrt(jnp.maximum(dr_ref[...][:, 0:1], 1.0))


def _row_spec(i_dim):
    return pl.BlockSpec((_BLK, i_dim), lambda i: (i, 0))


def _full_spec(r, c):
    return pl.BlockSpec((r, c), lambda i: (0, 0))


def _tc_layer1(x, W1, b1, degs):
    return pl.pallas_call(
        _l1_body,
        grid=(N // _BLK,),
        in_specs=[_row_spec(D), _full_spec(D, D), _full_spec(1, D), _row_spec(L)],
        out_specs=[_row_spec(DH), _row_spec(DH)],
        out_shape=[
            jax.ShapeDtypeStruct((N, DH), jnp.float32),
            jax.ShapeDtypeStruct((N, DH), jnp.float32),
        ],
    )(x, W1, b1.reshape(1, D), degs)


def _tc_layer2(plo, phi, hlo, hhi, degr, W2, b2, degs):
    return pl.pallas_call(
        _l2_body,
        grid=(N // _BLK,),
        in_specs=[
            _row_spec(DH),
            _row_spec(DH),
            _row_spec(DH),
            _row_spec(DH),
            _row_spec(L),
            _full_spec(D, D),
            _full_spec(1, D),
            _row_spec(L),
        ],
        out_specs=[_row_spec(DH), _row_spec(DH)],
        out_shape=[
            jax.ShapeDtypeStruct((N, DH), jnp.float32),
            jax.ShapeDtypeStruct((N, DH), jnp.float32),
        ],
    )(plo, phi, hlo, hhi, degr, W2, b2.reshape(1, D), degs)


def _tc_final(qlo, qhi, degr):
    return pl.pallas_call(
        _fin_body,
        grid=(N // _BLK,),
        in_specs=[_row_spec(DH), _row_spec(DH), _row_spec(L)],
        out_specs=_row_spec(D),
        out_shape=jax.ShapeDtypeStruct((N, D), jnp.float32),
    )(qlo, qhi, degr)


@jax.jit
def kernel(x, senders, receivers, W1, b1, W2, b2):
    senders = senders.astype(jnp.int32)
    receivers = receivers.astype(jnp.int32)

    sidx = senders.reshape(NS, NCH, CK)
    ridx = receivers.reshape(NS, NCH, CK)
    deg_in = jnp.stack([sidx, ridx])
    deg = _degree_call(deg_in)
    degs, degr = deg[0], deg[1]

    h1lo, h1hi = _tc_layer1(x, W1, b1, degs)
    plo, phi = _scatter_call(h1lo, h1hi, sidx, ridx)
    h2lo, h2hi = _tc_layer2(plo, phi, h1lo, h1hi, degr, W2, b2, degs)
    qlo, qhi = _scatter_call(h2lo, h2hi, sidx, ridx)
    return _tc_final(qlo, qhi, degr)


# speedup vs baseline: 10.3864x; 1.5166x over previous
import functools

import jax
import jax.numpy as jnp
from jax import lax
from jax.experimental import pallas as pl
from jax.experimental.pallas import tpu as pltpu
from jax.experimental.pallas import tpu_sc as plsc

N = 10000
E = 320000
D = 128
DH = D // 2
NC = 2
NS = 16
L = 16
CK = 125
NCH = (E // NS) // CK
STRIPE = 624
LAST_STRIPE = N - (NS - 1) * STRIPE

_MESH = plsc.VectorSubcoreMesh(
    core_axis_name="c", subcore_axis_name="s", num_cores=NC, num_subcores=NS
)
_SC_PARAMS = pltpu.CompilerParams(use_tc_tiling_on_sc=False)


def _stripe_out(acc, out_ref, s):
    base = s * STRIPE

    @pl.when(s < NS - 1)
    def _():
        pltpu.sync_copy(acc.at[pl.ds(base, STRIPE)], out_ref.at[pl.ds(base, STRIPE)])

    @pl.when(s == NS - 1)
    def _():
        pltpu.sync_copy(
            acc.at[pl.ds(base, LAST_STRIPE)], out_ref.at[pl.ds(base, LAST_STRIPE)]
        )


def _degree_body(idx_hbm, out_hbm, idx_v, ones_v, zb_v, acc):
    c = lax.axis_index("c")
    s = lax.axis_index("s")
    ones16 = jnp.full((L,), 1.0, jnp.float32)
    zeros16 = jnp.zeros((L,), jnp.float32)

    @pl.loop(0, CK)
    def _(j):
        ones_v[j] = ones16

    @pl.loop(0, LAST_STRIPE)
    def _(j):
        zb_v[j] = zeros16

    base = s * STRIPE
    pltpu.sync_copy(zb_v, acc.at[pl.ds(base, LAST_STRIPE)])
    plsc.subcore_barrier()

    pltpu.sync_copy(idx_hbm.at[c, s], idx_v)

    @pl.loop(0, NCH)
    def _(j):
        pltpu.sync_copy(ones_v, acc.at[idx_v.at[j]], add=True)

    plsc.subcore_barrier()
    _stripe_out(acc, out_hbm.at[c], s)


_degree_call = functools.partial(
    pl.kernel,
    out_type=jax.ShapeDtypeStruct((NC, N, L), jnp.float32),
    mesh=_MESH,
    scratch_types=[
        pltpu.VMEM((NCH, CK), jnp.int32),
        pltpu.VMEM((CK, L), jnp.float32),
        pltpu.VMEM((LAST_STRIPE, L), jnp.float32),
        pltpu.VMEM_SHARED((N, L), jnp.float32),
    ],
    compiler_params=_SC_PARAMS,
)(_degree_body)


def _scatter_body(
    hlo_hbm, hhi_hbm, sidx_hbm, ridx_hbm, olo_hbm, ohi_hbm,
    sidx_v, ridx_v, rows0_v, rows1_v, zb_v, acc, gsem0, gsem1,
):
    c = lax.axis_index("c")
    s = lax.axis_index("s")
    zeros16 = jnp.zeros((L,), jnp.float32)

    @pl.loop(0, LAST_STRIPE // 2)
    def _(j):
        for k in range(DH // L):
            zb_v[j, pl.ds(k * L, L)] = zeros16

    base = s * STRIPE
    pltpu.sync_copy(zb_v, acc.at[pl.ds(base, LAST_STRIPE // 2)])
    pltpu.sync_copy(zb_v, acc.at[pl.ds(base + LAST_STRIPE // 2, LAST_STRIPE // 2)])
    plsc.subcore_barrier()

    pltpu.sync_copy(sidx_hbm.at[s], sidx_v)
    pltpu.sync_copy(ridx_hbm.at[s], ridx_v)

    def gather(j, rows, gsem):
        @pl.when(c == 0)
        def _():
            pltpu.async_copy(hlo_hbm.at[sidx_v.at[j]], rows, gsem)

        @pl.when(c == 1)
        def _():
            pltpu.async_copy(hhi_hbm.at[sidx_v.at[j]], rows, gsem)

    def gather_wait(j, rows, gsem):
        pltpu.make_async_copy(hlo_hbm.at[sidx_v.at[j]], rows, gsem).wait()

    gather(0, rows0_v, gsem0)

    @pl.loop(0, NCH // 2)
    def _(g):
        a = g * 2
        gather(a + 1, rows1_v, gsem1)
        gather_wait(a, rows0_v, gsem0)
        pltpu.sync_copy(rows0_v, acc.at[ridx_v.at[a]], add=True)

        @pl.when(g < NCH // 2 - 1)
        def _():
            gather(a + 2, rows0_v, gsem0)

        gather_wait(a + 1, rows1_v, gsem1)
        pltpu.sync_copy(rows1_v, acc.at[ridx_v.at[a + 1]], add=True)

    plsc.subcore_barrier()

    @pl.when(c == 0)
    def _():
        _stripe_out(acc, olo_hbm, s)

    @pl.when(c == 1)
    def _():
        _stripe_out(acc, ohi_hbm, s)


_scatter_call = functools.partial(
    pl.kernel,
    out_type=[
        jax.ShapeDtypeStruct((N, DH), jnp.float32),
        jax.ShapeDtypeStruct((N, DH), jnp.float32),
    ],
    mesh=_MESH,
    scratch_types=[
        pltpu.VMEM((NCH, CK), jnp.int32),
        pltpu.VMEM((NCH, CK), jnp.int32),
        pltpu.VMEM((CK, DH), jnp.float32),
        pltpu.VMEM((CK, DH), jnp.float32),
        pltpu.VMEM((LAST_STRIPE // 2, DH), jnp.float32),
        pltpu.VMEM_SHARED((N, DH), jnp.float32),
        pltpu.SemaphoreType.DMA,
        pltpu.SemaphoreType.DMA,
    ],
    compiler_params=_SC_PARAMS,
)(_scatter_body)


_BLK = 1000


def _l1_body(x_ref, w_ref, b_ref, d_ref, olo_ref, ohi_ref):
    h = jnp.tanh(
        jnp.dot(x_ref[...], w_ref[...], preferred_element_type=jnp.float32)
        + b_ref[...]
    )
    dcol = d_ref[...][:, 0:1]
    h = h * lax.rsqrt(jnp.maximum(dcol + 1.0, 1.0))
    olo_ref[...] = h[:, :DH]
    ohi_ref[...] = h[:, DH:]


def _l2_body(plo_ref, phi_ref, hlo_ref, hhi_ref, dr_ref, w_ref, b_ref, ds_ref,
             olo_ref, ohi_ref):
    agg = jnp.concatenate(
        [plo_ref[...] + hlo_ref[...], phi_ref[...] + hhi_ref[...]], axis=1
    ) * lax.rsqrt(jnp.maximum(dr_ref[...][:, 0:1] + 1.0, 1.0))
    h2 = jnp.dot(agg, w_ref[...], preferred_element_type=jnp.float32) + b_ref[...]
    h2 = h2 * lax.rsqrt(jnp.maximum(ds_ref[...][:, 0:1], 1.0))
    olo_ref[...] = h2[:, :DH]
    ohi_ref[...] = h2[:, DH:]


def _fin_body(qlo_ref, qhi_ref, dr_ref, o_ref):
    o_ref[...] = jnp.concatenate(
        [qlo_ref[...], qhi_ref[...]], axis=1
    ) * lax.rsqrt(jnp.maximum(dr_ref[...][:, 0:1], 1.0))


def _row_spec(i_dim):
    return pl.BlockSpec((_BLK, i_dim), lambda i: (i, 0))


def _full_spec(r, c):
    return pl.BlockSpec((r, c), lambda i: (0, 0))


def _tc_layer1(x, W1, b1, degs):
    return pl.pallas_call(
        _l1_body,
        grid=(N // _BLK,),
        in_specs=[_row_spec(D), _full_spec(D, D), _full_spec(1, D), _row_spec(L)],
        out_specs=[_row_spec(DH), _row_spec(DH)],
        out_shape=[
            jax.ShapeDtypeStruct((N, DH), jnp.float32),
            jax.ShapeDtypeStruct((N, DH), jnp.float32),
        ],
    )(x, W1, b1.reshape(1, D), degs)


def _tc_layer2(plo, phi, hlo, hhi, degr, W2, b2, degs):
    return pl.pallas_call(
        _l2_body,
        grid=(N // _BLK,),
        in_specs=[
            _row_spec(DH),
            _row_spec(DH),
            _row_spec(DH),
            _row_spec(DH),
            _row_spec(L),
            _full_spec(D, D),
            _full_spec(1, D),
            _row_spec(L),
        ],
        out_specs=[_row_spec(DH), _row_spec(DH)],
        out_shape=[
            jax.ShapeDtypeStruct((N, DH), jnp.float32),
            jax.ShapeDtypeStruct((N, DH), jnp.float32),
        ],
    )(plo, phi, hlo, hhi, degr, W2, b2.reshape(1, D), degs)


def _tc_final(qlo, qhi, degr):
    return pl.pallas_call(
        _fin_body,
        grid=(N // _BLK,),
        in_specs=[_row_spec(DH), _row_spec(DH), _row_spec(L)],
        out_specs=_row_spec(D),
        out_shape=jax.ShapeDtypeStruct((N, D), jnp.float32),
    )(qlo, qhi, degr)


@jax.jit
def kernel(x, senders, receivers, W1, b1, W2, b2):
    senders = senders.astype(jnp.int32)
    receivers = receivers.astype(jnp.int32)

    sidx = senders.reshape(NS, NCH, CK)
    ridx = receivers.reshape(NS, NCH, CK)
    deg_in = jnp.stack([sidx, ridx])
    deg = _degree_call(deg_in)
    degs, degr = deg[0], deg[1]

    h1lo, h1hi = _tc_layer1(x, W1, b1, degs)
    plo, phi = _scatter_call(h1lo, h1hi, sidx, ridx)
    h2lo, h2hi = _tc_layer2(plo, phi, h1lo, h1hi, degr, W2, b2, degs)
    qlo, qhi = _scatter_call(h2lo, h2hi, sidx, ridx)
    return _tc_final(qlo, qhi, degr)


# speedup vs baseline: 12.6252x; 1.2156x over previous
import functools

import jax
import jax.numpy as jnp
from jax import lax
from jax.experimental import pallas as pl
from jax.experimental.pallas import tpu as pltpu
from jax.experimental.pallas import tpu_sc as plsc

N = 10000
E = 320000
D = 128
DH = D // 2
NC = 2
NS = 16
L = 16
CK = 125
NCH = (E // NS) // CK
STRIPE = 624
LAST_STRIPE = N - (NS - 1) * STRIPE

_MESH = plsc.VectorSubcoreMesh(
    core_axis_name="c", subcore_axis_name="s", num_cores=NC, num_subcores=NS
)
_SC_PARAMS = pltpu.CompilerParams(use_tc_tiling_on_sc=False)


def _zero_stripe(zsrc, acc, base):
    for k in range(5):
        pltpu.sync_copy(zsrc.at[pl.ds(0, 120)], acc.at[pl.ds(base + 120 * k, 120)])
    pltpu.sync_copy(zsrc.at[pl.ds(0, 40)], acc.at[pl.ds(base + 600, 40)])


def _stripe_out(acc, out_ref, s):
    base = s * STRIPE

    @pl.when(s < NS - 1)
    def _():
        pltpu.sync_copy(acc.at[pl.ds(base, STRIPE)], out_ref.at[pl.ds(base, STRIPE)])

    @pl.when(s == NS - 1)
    def _():
        pltpu.sync_copy(
            acc.at[pl.ds(base, LAST_STRIPE)], out_ref.at[pl.ds(base, LAST_STRIPE)]
        )


def _degree_body(idx_hbm, out_hbm, idx_v, ones_v, zb_v, acc):
    c = lax.axis_index("c")
    s = lax.axis_index("s")
    ones16 = jnp.full((L,), 1.0, jnp.float32)
    zeros16 = jnp.zeros((L,), jnp.float32)

    @pl.loop(0, CK)
    def _(j):
        ones_v[j] = ones16

    @pl.loop(0, 120)
    def _(j):
        zb_v[j] = zeros16

    base = s * STRIPE
    _zero_stripe(zb_v, acc, base)
    plsc.subcore_barrier()

    pltpu.sync_copy(idx_hbm.at[c, s], idx_v)

    @pl.loop(0, NCH)
    def _(j):
        pltpu.sync_copy(ones_v, acc.at[idx_v.at[j]], add=True)

    plsc.subcore_barrier()
    _stripe_out(acc, out_hbm.at[c], s)


_degree_call = functools.partial(
    pl.kernel,
    out_type=jax.ShapeDtypeStruct((NC, N, L), jnp.float32),
    mesh=_MESH,
    scratch_types=[
        pltpu.VMEM((NCH, CK), jnp.int32),
        pltpu.VMEM((CK, L), jnp.float32),
        pltpu.VMEM((120, L), jnp.float32),
        pltpu.VMEM_SHARED((N, L), jnp.float32),
    ],
    compiler_params=_SC_PARAMS,
)(_degree_body)


NBUF = 4


def _scatter_body(
    hlo_hbm, hhi_hbm, sidx_hbm, ridx_hbm, olo_hbm, ohi_hbm,
    sidx_v, ridx_v, rb0, rb1, rb2, rb3, acc, gs0, gs1, gs2, gs3,
):
    rows_bufs = (rb0, rb1, rb2, rb3)
    gsems = (gs0, gs1, gs2, gs3)
    c = lax.axis_index("c")
    s = lax.axis_index("s")
    zeros16 = jnp.zeros((L,), jnp.float32)

    @pl.loop(0, CK)
    def _(j):
        for k in range(DH // L):
            rb0[j, pl.ds(k * L, L)] = zeros16

    base = s * STRIPE
    _zero_stripe(rb0, acc, base)
    plsc.subcore_barrier()

    pltpu.sync_copy(sidx_hbm.at[s], sidx_v)
    pltpu.sync_copy(ridx_hbm.at[s], ridx_v)

    def gather(j, rows, gsem):
        @pl.when(c == 0)
        def _():
            pltpu.async_copy(hlo_hbm.at[sidx_v.at[j]], rows, gsem)

        @pl.when(c == 1)
        def _():
            pltpu.async_copy(hhi_hbm.at[sidx_v.at[j]], rows, gsem)

    def gather_wait(j, rows, gsem):
        pltpu.make_async_copy(hlo_hbm.at[sidx_v.at[j]], rows, gsem).wait()

    for b in range(NBUF):
        gather(b, rows_bufs[b], gsems[b])

    @pl.loop(0, NCH // NBUF)
    def _(g):
        a = g * NBUF
        for b in range(NBUF):
            j = a + b
            gather_wait(j, rows_bufs[b], gsems[b])
            pltpu.sync_copy(rows_bufs[b], acc.at[ridx_v.at[j]], add=True)

            @pl.when(g < NCH // NBUF - 1)
            def _():
                gather(j + NBUF, rows_bufs[b], gsems[b])

    plsc.subcore_barrier()

    @pl.when(c == 0)
    def _():
        _stripe_out(acc, olo_hbm, s)

    @pl.when(c == 1)
    def _():
        _stripe_out(acc, ohi_hbm, s)


_scatter_call = functools.partial(
    pl.kernel,
    out_type=[
        jax.ShapeDtypeStruct((N, DH), jnp.float32),
        jax.ShapeDtypeStruct((N, DH), jnp.float32),
    ],
    mesh=_MESH,
    scratch_types=[
        pltpu.VMEM((NCH, CK), jnp.int32),
        pltpu.VMEM((NCH, CK), jnp.int32),
        pltpu.VMEM((CK, DH), jnp.float32),
        pltpu.VMEM((CK, DH), jnp.float32),
        pltpu.VMEM((CK, DH), jnp.float32),
        pltpu.VMEM((CK, DH), jnp.float32),
        pltpu.VMEM_SHARED((N, DH), jnp.float32),
        pltpu.SemaphoreType.DMA,
        pltpu.SemaphoreType.DMA,
        pltpu.SemaphoreType.DMA,
        pltpu.SemaphoreType.DMA,
    ],
    compiler_params=_SC_PARAMS,
)(_scatter_body)


_BLK = 1000


def _l1_body(x_ref, w_ref, b_ref, d_ref, olo_ref, ohi_ref):
    h = jnp.tanh(
        jnp.dot(x_ref[...], w_ref[...], preferred_element_type=jnp.float32)
        + b_ref[...]
    )
    dcol = d_ref[...][:, 0:1]
    h = h * lax.rsqrt(jnp.maximum(dcol + 1.0, 1.0))
    olo_ref[...] = h[:, :DH]
    ohi_ref[...] = h[:, DH:]


def _l2_body(plo_ref, phi_ref, hlo_ref, hhi_ref, dr_ref, w_ref, b_ref, ds_ref,
             olo_ref, ohi_ref):
    agg = jnp.concatenate(
        [plo_ref[...] + hlo_ref[...], phi_ref[...] + hhi_ref[...]], axis=1
    ) * lax.rsqrt(jnp.maximum(dr_ref[...][:, 0:1] + 1.0, 1.0))
    h2 = jnp.dot(agg, w_ref[...], preferred_element_type=jnp.float32) + b_ref[...]
    h2 = h2 * lax.rsqrt(jnp.maximum(ds_ref[...][:, 0:1], 1.0))
    olo_ref[...] = h2[:, :DH]
    ohi_ref[...] = h2[:, DH:]


def _fin_body(qlo_ref, qhi_ref, dr_ref, o_ref):
    o_ref[...] = jnp.concatenate(
        [qlo_ref[...], qhi_ref[...]], axis=1
    ) * lax.rsqrt(jnp.maximum(dr_ref[...][:, 0:1], 1.0))


def _row_spec(i_dim):
    return pl.BlockSpec((_BLK, i_dim), lambda i: (i, 0))


def _full_spec(r, c):
    return pl.BlockSpec((r, c), lambda i: (0, 0))


def _tc_layer1(x, W1, b1, degs):
    return pl.pallas_call(
        _l1_body,
        grid=(N // _BLK,),
        in_specs=[_row_spec(D), _full_spec(D, D), _full_spec(1, D), _row_spec(L)],
        out_specs=[_row_spec(DH), _row_spec(DH)],
        out_shape=[
            jax.ShapeDtypeStruct((N, DH), jnp.float32),
            jax.ShapeDtypeStruct((N, DH), jnp.float32),
        ],
    )(x, W1, b1.reshape(1, D), degs)


def _tc_layer2(plo, phi, hlo, hhi, degr, W2, b2, degs):
    return pl.pallas_call(
        _l2_body,
        grid=(N // _BLK,),
        in_specs=[
            _row_spec(DH),
            _row_spec(DH),
            _row_spec(DH),
            _row_spec(DH),
            _row_spec(L),
            _full_spec(D, D),
            _full_spec(1, D),
            _row_spec(L),
        ],
        out_specs=[_row_spec(DH), _row_spec(DH)],
        out_shape=[
            jax.ShapeDtypeStruct((N, DH), jnp.float32),
            jax.ShapeDtypeStruct((N, DH), jnp.float32),
        ],
    )(plo, phi, hlo, hhi, degr, W2, b2.reshape(1, D), degs)


def _tc_final(qlo, qhi, degr):
    return pl.pallas_call(
        _fin_body,
        grid=(N // _BLK,),
        in_specs=[_row_spec(DH), _row_spec(DH), _row_spec(L)],
        out_specs=_row_spec(D),
        out_shape=jax.ShapeDtypeStruct((N, D), jnp.float32),
    )(qlo, qhi, degr)


@jax.jit
def kernel(x, senders, receivers, W1, b1, W2, b2):
    senders = senders.astype(jnp.int32)
    receivers = receivers.astype(jnp.int32)

    sidx = senders.reshape(NS, NCH, CK)
    ridx = receivers.reshape(NS, NCH, CK)
    deg_in = jnp.stack([sidx, ridx])
    deg = _degree_call(deg_in)
    degs, degr = deg[0], deg[1]

    h1lo, h1hi = _tc_layer1(x, W1, b1, degs)
    plo, phi = _scatter_call(h1lo, h1hi, sidx, ridx)
    h2lo, h2hi = _tc_layer2(plo, phi, h1lo, h1hi, degr, W2, b2, degs)
    qlo, qhi = _scatter_call(h2lo, h2hi, sidx, ridx)
    return _tc_final(qlo, qhi, degr)


# speedup vs baseline: 12.8250x; 1.0158x over previous
import functools

import jax
import jax.numpy as jnp
from jax import lax
from jax.experimental import pallas as pl
from jax.experimental.pallas import tpu as pltpu
from jax.experimental.pallas import tpu_sc as plsc

N = 10000
E = 320000
D = 128
DH = D // 2
NC = 2
NS = 16
L = 16
CK = 125
NCH = (E // NS) // CK
STRIPE = 624
LAST_STRIPE = N - (NS - 1) * STRIPE

_MESH = plsc.VectorSubcoreMesh(
    core_axis_name="c", subcore_axis_name="s", num_cores=NC, num_subcores=NS
)
_SC_PARAMS = pltpu.CompilerParams(use_tc_tiling_on_sc=False)


def _zero_stripe(zsrc, acc, base):
    for k in range(5):
        pltpu.sync_copy(zsrc.at[pl.ds(0, 120)], acc.at[pl.ds(base + 120 * k, 120)])
    pltpu.sync_copy(zsrc.at[pl.ds(0, 40)], acc.at[pl.ds(base + 600, 40)])


def _stripe_out(acc, out_ref, s):
    base = s * STRIPE

    @pl.when(s < NS - 1)
    def _():
        pltpu.sync_copy(acc.at[pl.ds(base, STRIPE)], out_ref.at[pl.ds(base, STRIPE)])

    @pl.when(s == NS - 1)
    def _():
        pltpu.sync_copy(
            acc.at[pl.ds(base, LAST_STRIPE)], out_ref.at[pl.ds(base, LAST_STRIPE)]
        )


def _degree_body(idx_hbm, out_hbm, idx_v, ones_v, zb_v, acc):
    c = lax.axis_index("c")
    s = lax.axis_index("s")
    ones16 = jnp.full((L,), 1.0, jnp.float32)
    zeros16 = jnp.zeros((L,), jnp.float32)

    @pl.loop(0, CK)
    def _(j):
        ones_v[j] = ones16

    @pl.loop(0, 120)
    def _(j):
        zb_v[j] = zeros16

    base = s * STRIPE
    _zero_stripe(zb_v, acc, base)
    plsc.subcore_barrier()

    pltpu.sync_copy(idx_hbm.at[c, s], idx_v)

    @pl.loop(0, NCH)
    def _(j):
        pltpu.sync_copy(ones_v, acc.at[idx_v.at[j]], add=True)

    plsc.subcore_barrier()
    _stripe_out(acc, out_hbm.at[c], s)


_degree_call = functools.partial(
    pl.kernel,
    out_type=jax.ShapeDtypeStruct((NC, N, L), jnp.float32),
    mesh=_MESH,
    scratch_types=[
        pltpu.VMEM((NCH, CK), jnp.int32),
        pltpu.VMEM((CK, L), jnp.float32),
        pltpu.VMEM((120, L), jnp.float32),
        pltpu.VMEM_SHARED((N, L), jnp.float32),
    ],
    compiler_params=_SC_PARAMS,
)(_degree_body)


NBUF = 6


def _scatter_body(
    hlo_hbm, hhi_hbm, sidx_hbm, ridx_hbm, olo_hbm, ohi_hbm,
    sidx_v, ridx_v, rb0, rb1, rb2, rb3, rb4, rb5, acc,
    gs0, gs1, gs2, gs3, gs4, gs5, isem,
):
    rows_bufs = (rb0, rb1, rb2, rb3, rb4, rb5)
    gsems = (gs0, gs1, gs2, gs3, gs4, gs5)
    c = lax.axis_index("c")
    s = lax.axis_index("s")
    zeros16 = jnp.zeros((L,), jnp.float32)

    pltpu.async_copy(sidx_hbm.at[s], sidx_v, isem)
    pltpu.async_copy(ridx_hbm.at[s], ridx_v, isem)

    @pl.loop(0, CK)
    def _(j):
        for k in range(DH // L):
            rb0[j, pl.ds(k * L, L)] = zeros16

    base = s * STRIPE
    _zero_stripe(rb0, acc, base)
    pltpu.make_async_copy(sidx_hbm.at[s], sidx_v, isem).wait()
    pltpu.make_async_copy(ridx_hbm.at[s], ridx_v, isem).wait()
    plsc.subcore_barrier()

    def gather(j, rows, gsem):
        @pl.when(c == 0)
        def _():
            pltpu.async_copy(hlo_hbm.at[sidx_v.at[j]], rows, gsem)

        @pl.when(c == 1)
        def _():
            pltpu.async_copy(hhi_hbm.at[sidx_v.at[j]], rows, gsem)

    def gather_wait(j, rows, gsem):
        pltpu.make_async_copy(hlo_hbm.at[sidx_v.at[j]], rows, gsem).wait()

    for b in range(NBUF):
        gather(b, rows_bufs[b], gsems[b])

    NFULL = NCH // NBUF
    NREM = NCH - NFULL * NBUF

    @pl.loop(0, NFULL)
    def _(g):
        a = g * NBUF
        for b in range(NBUF):
            j = a + b
            gather_wait(j, rows_bufs[b], gsems[b])
            pltpu.sync_copy(rows_bufs[b], acc.at[ridx_v.at[j]], add=True)
            jn = j + NBUF

            @pl.when(jn < NCH)
            def _():
                gather(jn, rows_bufs[b], gsems[b])

    for b in range(NREM):
        j = NFULL * NBUF + b
        gather_wait(j, rows_bufs[b], gsems[b])
        pltpu.sync_copy(rows_bufs[b], acc.at[ridx_v.at[j]], add=True)

    plsc.subcore_barrier()

    @pl.when(c == 0)
    def _():
        _stripe_out(acc, olo_hbm, s)

    @pl.when(c == 1)
    def _():
        _stripe_out(acc, ohi_hbm, s)


_scatter_call = functools.partial(
    pl.kernel,
    out_type=[
        jax.ShapeDtypeStruct((N, DH), jnp.float32),
        jax.ShapeDtypeStruct((N, DH), jnp.float32),
    ],
    mesh=_MESH,
    scratch_types=[
        pltpu.VMEM((NCH, CK), jnp.int32),
        pltpu.VMEM((NCH, CK), jnp.int32),
        pltpu.VMEM((CK, DH), jnp.float32),
        pltpu.VMEM((CK, DH), jnp.float32),
        pltpu.VMEM((CK, DH), jnp.float32),
        pltpu.VMEM((CK, DH), jnp.float32),
        pltpu.VMEM((CK, DH), jnp.float32),
        pltpu.VMEM((CK, DH), jnp.float32),
        pltpu.VMEM_SHARED((N, DH), jnp.float32),
        pltpu.SemaphoreType.DMA,
        pltpu.SemaphoreType.DMA,
        pltpu.SemaphoreType.DMA,
        pltpu.SemaphoreType.DMA,
        pltpu.SemaphoreType.DMA,
        pltpu.SemaphoreType.DMA,
        pltpu.SemaphoreType.DMA,
    ],
    compiler_params=_SC_PARAMS,
)(_scatter_body)


_BLK = 1000
_TC_PARAMS = pltpu.CompilerParams(dimension_semantics=("parallel",))


def _l1_body(x_ref, w_ref, b_ref, d_ref, olo_ref, ohi_ref):
    h = jnp.tanh(
        jnp.dot(x_ref[...], w_ref[...], preferred_element_type=jnp.float32)
        + b_ref[...]
    )
    dcol = d_ref[...][:, 0:1]
    h = h * lax.rsqrt(jnp.maximum(dcol + 1.0, 1.0))
    olo_ref[...] = h[:, :DH]
    ohi_ref[...] = h[:, DH:]


def _l2_body(plo_ref, phi_ref, hlo_ref, hhi_ref, dr_ref, w_ref, b_ref, ds_ref,
             olo_ref, ohi_ref):
    agg = jnp.concatenate(
        [plo_ref[...] + hlo_ref[...], phi_ref[...] + hhi_ref[...]], axis=1
    ) * lax.rsqrt(jnp.maximum(dr_ref[...][:, 0:1] + 1.0, 1.0))
    h2 = jnp.dot(agg, w_ref[...], preferred_element_type=jnp.float32) + b_ref[...]
    h2 = h2 * lax.rsqrt(jnp.maximum(ds_ref[...][:, 0:1], 1.0))
    olo_ref[...] = h2[:, :DH]
    ohi_ref[...] = h2[:, DH:]


def _fin_body(qlo_ref, qhi_ref, dr_ref, o_ref):
    o_ref[...] = jnp.concatenate(
        [qlo_ref[...], qhi_ref[...]], axis=1
    ) * lax.rsqrt(jnp.maximum(dr_ref[...][:, 0:1], 1.0))


def _row_spec(i_dim):
    return pl.BlockSpec((_BLK, i_dim), lambda i: (i, 0))


def _full_spec(r, c):
    return pl.BlockSpec((r, c), lambda i: (0, 0))


def _tc_layer1(x, W1, b1, degs):
    return pl.pallas_call(
        _l1_body,
        grid=(N // _BLK,),
        compiler_params=_TC_PARAMS,
        in_specs=[_row_spec(D), _full_spec(D, D), _full_spec(1, D), _row_spec(L)],
        out_specs=[_row_spec(DH), _row_spec(DH)],
        out_shape=[
            jax.ShapeDtypeStruct((N, DH), jnp.float32),
            jax.ShapeDtypeStruct((N, DH), jnp.float32),
        ],
    )(x, W1, b1.reshape(1, D), degs)


def _tc_layer2(plo, phi, hlo, hhi, degr, W2, b2, degs):
    return pl.pallas_call(
        _l2_body,
        grid=(N // _BLK,),
        compiler_params=_TC_PARAMS,
        in_specs=[
            _row_spec(DH),
            _row_spec(DH),
            _row_spec(DH),
            _row_spec(DH),
            _row_spec(L),
            _full_spec(D, D),
            _full_spec(1, D),
            _row_spec(L),
        ],
        out_specs=[_row_spec(DH), _row_spec(DH)],
        out_shape=[
            jax.ShapeDtypeStruct((N, DH), jnp.float32),
            jax.ShapeDtypeStruct((N, DH), jnp.float32),
        ],
    )(plo, phi, hlo, hhi, degr, W2, b2.reshape(1, D), degs)


def _tc_final(qlo, qhi, degr):
    return pl.pallas_call(
        _fin_body,
        grid=(N // _BLK,),
        compiler_params=_TC_PARAMS,
        in_specs=[_row_spec(DH), _row_spec(DH), _row_spec(L)],
        out_specs=_row_spec(D),
        out_shape=jax.ShapeDtypeStruct((N, D), jnp.float32),
    )(qlo, qhi, degr)


@jax.jit
def kernel(x, senders, receivers, W1, b1, W2, b2):
    senders = senders.astype(jnp.int32)
    receivers = receivers.astype(jnp.int32)

    sidx = senders.reshape(NS, NCH, CK)
    ridx = receivers.reshape(NS, NCH, CK)
    deg_in = jnp.stack([sidx, ridx])
    deg = _degree_call(deg_in)
    degs, degr = deg[0], deg[1]

    h1lo, h1hi = _tc_layer1(x, W1, b1, degs)
    plo, phi = _scatter_call(h1lo, h1hi, sidx, ridx)
    h2lo, h2hi = _tc_layer2(plo, phi, h1lo, h1hi, degr, W2, b2, degs)
    qlo, qhi = _scatter_call(h2lo, h2hi, sidx, ridx)
    return _tc_final(qlo, qhi, degr)


# speedup vs baseline: 12.8331x; 1.0006x over previous
import functools

import jax
import jax.numpy as jnp
from jax import lax
from jax.experimental import pallas as pl
from jax.experimental.pallas import tpu as pltpu
from jax.experimental.pallas import tpu_sc as plsc

N = 10000
E = 320000
D = 128
DH = D // 2
NC = 2
NS = 16
L = 16
CK = 125
NCH = (E // NS) // CK
STRIPE = 624
LAST_STRIPE = N - (NS - 1) * STRIPE

_MESH = plsc.VectorSubcoreMesh(
    core_axis_name="c", subcore_axis_name="s", num_cores=NC, num_subcores=NS
)
_SC_PARAMS = pltpu.CompilerParams(use_tc_tiling_on_sc=False, skip_device_barrier=True)


def _zero_stripe(zsrc, acc, base):
    for k in range(5):
        pltpu.sync_copy(zsrc.at[pl.ds(0, 120)], acc.at[pl.ds(base + 120 * k, 120)])
    pltpu.sync_copy(zsrc.at[pl.ds(0, 40)], acc.at[pl.ds(base + 600, 40)])


def _stripe_out(acc, out_ref, s):
    base = s * STRIPE

    @pl.when(s < NS - 1)
    def _():
        pltpu.sync_copy(acc.at[pl.ds(base, STRIPE)], out_ref.at[pl.ds(base, STRIPE)])

    @pl.when(s == NS - 1)
    def _():
        pltpu.sync_copy(
            acc.at[pl.ds(base, LAST_STRIPE)], out_ref.at[pl.ds(base, LAST_STRIPE)]
        )


def _degree_body(idx_hbm, out_hbm, idx_v, ones_v, zb_v, acc):
    c = lax.axis_index("c")
    s = lax.axis_index("s")
    ones16 = jnp.full((L,), 1.0, jnp.float32)
    zeros16 = jnp.zeros((L,), jnp.float32)

    @pl.loop(0, CK)
    def _(j):
        ones_v[j] = ones16

    @pl.loop(0, 120)
    def _(j):
        zb_v[j] = zeros16

    base = s * STRIPE
    _zero_stripe(zb_v, acc, base)
    plsc.subcore_barrier()

    pltpu.sync_copy(idx_hbm.at[c, s], idx_v)

    @pl.loop(0, NCH)
    def _(j):
        pltpu.sync_copy(ones_v, acc.at[idx_v.at[j]], add=True)

    plsc.subcore_barrier()
    _stripe_out(acc, out_hbm.at[c], s)


_degree_call = functools.partial(
    pl.kernel,
    out_type=jax.ShapeDtypeStruct((NC, N, L), jnp.float32),
    mesh=_MESH,
    scratch_types=[
        pltpu.VMEM((NCH, CK), jnp.int32),
        pltpu.VMEM((CK, L), jnp.float32),
        pltpu.VMEM((120, L), jnp.float32),
        pltpu.VMEM_SHARED((N, L), jnp.float32),
    ],
    compiler_params=_SC_PARAMS,
)(_degree_body)


NBUF = 6


def _scatter_body(
    hlo_hbm, hhi_hbm, sidx_hbm, ridx_hbm, olo_hbm, ohi_hbm,
    sidx_v, ridx_v, rb0, rb1, rb2, rb3, rb4, rb5, acc,
    gs0, gs1, gs2, gs3, gs4, gs5, isem,
):
    rows_bufs = (rb0, rb1, rb2, rb3, rb4, rb5)
    gsems = (gs0, gs1, gs2, gs3, gs4, gs5)
    c = lax.axis_index("c")
    s = lax.axis_index("s")
    zeros16 = jnp.zeros((L,), jnp.float32)

    pltpu.async_copy(sidx_hbm.at[s], sidx_v, isem)
    pltpu.async_copy(ridx_hbm.at[s], ridx_v, isem)

    @pl.loop(0, CK)
    def _(j):
        for k in range(DH // L):
            rb0[j, pl.ds(k * L, L)] = zeros16

    base = s * STRIPE
    _zero_stripe(rb0, acc, base)
    pltpu.make_async_copy(sidx_hbm.at[s], sidx_v, isem).wait()
    pltpu.make_async_copy(ridx_hbm.at[s], ridx_v, isem).wait()
    plsc.subcore_barrier()

    def gather(j, rows, gsem):
        @pl.when(c == 0)
        def _():
            pltpu.async_copy(hlo_hbm.at[sidx_v.at[j]], rows, gsem)

        @pl.when(c == 1)
        def _():
            pltpu.async_copy(hhi_hbm.at[sidx_v.at[j]], rows, gsem)

    def gather_wait(j, rows, gsem):
        pltpu.make_async_copy(hlo_hbm.at[sidx_v.at[j]], rows, gsem).wait()

    for b in range(NBUF):
        gather(b, rows_bufs[b], gsems[b])

    NFULL = NCH // NBUF
    NREM = NCH - NFULL * NBUF

    @pl.loop(0, NFULL)
    def _(g):
        a = g * NBUF
        for b in range(NBUF):
            j = a + b
            gather_wait(j, rows_bufs[b], gsems[b])
            pltpu.sync_copy(rows_bufs[b], acc.at[ridx_v.at[j]], add=True)
            jn = j + NBUF

            @pl.when(jn < NCH)
            def _():
                gather(jn, rows_bufs[b], gsems[b])

    for b in range(NREM):
        j = NFULL * NBUF + b
        gather_wait(j, rows_bufs[b], gsems[b])
        pltpu.sync_copy(rows_bufs[b], acc.at[ridx_v.at[j]], add=True)

    plsc.subcore_barrier()

    @pl.when(c == 0)
    def _():
        _stripe_out(acc, olo_hbm, s)

    @pl.when(c == 1)
    def _():
        _stripe_out(acc, ohi_hbm, s)


_scatter_call = functools.partial(
    pl.kernel,
    out_type=[
        jax.ShapeDtypeStruct((N, DH), jnp.float32),
        jax.ShapeDtypeStruct((N, DH), jnp.float32),
    ],
    mesh=_MESH,
    scratch_types=[
        pltpu.VMEM((NCH, CK), jnp.int32),
        pltpu.VMEM((NCH, CK), jnp.int32),
        pltpu.VMEM((CK, DH), jnp.float32),
        pltpu.VMEM((CK, DH), jnp.float32),
        pltpu.VMEM((CK, DH), jnp.float32),
        pltpu.VMEM((CK, DH), jnp.float32),
        pltpu.VMEM((CK, DH), jnp.float32),
        pltpu.VMEM((CK, DH), jnp.float32),
        pltpu.VMEM_SHARED((N, DH), jnp.float32),
        pltpu.SemaphoreType.DMA,
        pltpu.SemaphoreType.DMA,
        pltpu.SemaphoreType.DMA,
        pltpu.SemaphoreType.DMA,
        pltpu.SemaphoreType.DMA,
        pltpu.SemaphoreType.DMA,
        pltpu.SemaphoreType.DMA,
    ],
    compiler_params=_SC_PARAMS,
)(_scatter_body)


_BLK = 1000
_TC_PARAMS = pltpu.CompilerParams(dimension_semantics=("parallel",))


def _l1_body(x_ref, w_ref, b_ref, d_ref, olo_ref, ohi_ref):
    h = jnp.tanh(
        jnp.dot(x_ref[...], w_ref[...], preferred_element_type=jnp.float32)
        + b_ref[...]
    )
    dcol = d_ref[...][:, 0:1]
    h = h * lax.rsqrt(jnp.maximum(dcol + 1.0, 1.0))
    olo_ref[...] = h[:, :DH]
    ohi_ref[...] = h[:, DH:]


def _l2_body(plo_ref, phi_ref, hlo_ref, hhi_ref, dr_ref, w_ref, b_ref, ds_ref,
             olo_ref, ohi_ref):
    agg = jnp.concatenate(
        [plo_ref[...] + hlo_ref[...], phi_ref[...] + hhi_ref[...]], axis=1
    ) * lax.rsqrt(jnp.maximum(dr_ref[...][:, 0:1] + 1.0, 1.0))
    h2 = jnp.dot(agg, w_ref[...], preferred_element_type=jnp.float32) + b_ref[...]
    h2 = h2 * lax.rsqrt(jnp.maximum(ds_ref[...][:, 0:1], 1.0))
    olo_ref[...] = h2[:, :DH]
    ohi_ref[...] = h2[:, DH:]


def _fin_body(qlo_ref, qhi_ref, dr_ref, o_ref):
    o_ref[...] = jnp.concatenate(
        [qlo_ref[...], qhi_ref[...]], axis=1
    ) * lax.rsqrt(jnp.maximum(dr_ref[...][:, 0:1], 1.0))


def _row_spec(i_dim):
    return pl.BlockSpec((_BLK, i_dim), lambda i: (i, 0))


def _full_spec(r, c):
    return pl.BlockSpec((r, c), lambda i: (0, 0))


def _tc_layer1(x, W1, b1, degs):
    return pl.pallas_call(
        _l1_body,
        grid=(N // _BLK,),
        compiler_params=_TC_PARAMS,
        in_specs=[_row_spec(D), _full_spec(D, D), _full_spec(1, D), _row_spec(L)],
        out_specs=[_row_spec(DH), _row_spec(DH)],
        out_shape=[
            jax.ShapeDtypeStruct((N, DH), jnp.float32),
            jax.ShapeDtypeStruct((N, DH), jnp.float32),
        ],
    )(x, W1, b1.reshape(1, D), degs)


def _tc_layer2(plo, phi, hlo, hhi, degr, W2, b2, degs):
    return pl.pallas_call(
        _l2_body,
        grid=(N // _BLK,),
        compiler_params=_TC_PARAMS,
        in_specs=[
            _row_spec(DH),
            _row_spec(DH),
            _row_spec(DH),
            _row_spec(DH),
            _row_spec(L),
            _full_spec(D, D),
            _full_spec(1, D),
            _row_spec(L),
        ],
        out_specs=[_row_spec(DH), _row_spec(DH)],
        out_shape=[
            jax.ShapeDtypeStruct((N, DH), jnp.float32),
            jax.ShapeDtypeStruct((N, DH), jnp.float32),
        ],
    )(plo, phi, hlo, hhi, degr, W2, b2.reshape(1, D), degs)


def _tc_final(qlo, qhi, degr):
    return pl.pallas_call(
        _fin_body,
        grid=(N // _BLK,),
        compiler_params=_TC_PARAMS,
        in_specs=[_row_spec(DH), _row_spec(DH), _row_spec(L)],
        out_specs=_row_spec(D),
        out_shape=jax.ShapeDtypeStruct((N, D), jnp.float32),
    )(qlo, qhi, degr)


@jax.jit
def kernel(x, senders, receivers, W1, b1, W2, b2):
    senders = senders.astype(jnp.int32)
    receivers = receivers.astype(jnp.int32)

    sidx = senders.reshape(NS, NCH, CK)
    ridx = receivers.reshape(NS, NCH, CK)
    deg_in = jnp.stack([sidx, ridx])
    deg = _degree_call(deg_in)
    degs, degr = deg[0], deg[1]

    h1lo, h1hi = _tc_layer1(x, W1, b1, degs)
    plo, phi = _scatter_call(h1lo, h1hi, sidx, ridx)
    h2lo, h2hi = _tc_layer2(plo, phi, h1lo, h1hi, degr, W2, b2, degs)
    qlo, qhi = _scatter_call(h2lo, h2hi, sidx, ridx)
    return _tc_final(qlo, qhi, degr)


# speedup vs baseline: 13.3417x; 1.0396x over previous
import functools

import jax
import jax.numpy as jnp
from jax import lax
from jax.experimental import pallas as pl
from jax.experimental.pallas import tpu as pltpu
from jax.experimental.pallas import tpu_sc as plsc

N = 10000
E = 320000
D = 128
DH = D // 2
NC = 2
NS = 16
L = 16
CK = 125
NCH = (E // NS) // CK
STRIPE = 624
LAST_STRIPE = N - (NS - 1) * STRIPE
NP = 10240

_MESH = plsc.VectorSubcoreMesh(
    core_axis_name="c", subcore_axis_name="s", num_cores=NC, num_subcores=NS
)
_SC_PARAMS = pltpu.CompilerParams(use_tc_tiling_on_sc=False, skip_device_barrier=True)
_SC_PARAMS_NL = pltpu.CompilerParams(
    use_tc_tiling_on_sc=False, skip_device_barrier=True, needs_layout_passes=False
)


def _zero_stripe(zsrc, acc, base):
    for k in range(5):
        pltpu.sync_copy(zsrc.at[pl.ds(0, 120)], acc.at[pl.ds(base + 120 * k, 120)])
    pltpu.sync_copy(zsrc.at[pl.ds(0, 40)], acc.at[pl.ds(base + 600, 40)])


def _stripe_out(acc, out_ref, s):
    base = s * STRIPE

    @pl.when(s < NS - 1)
    def _():
        pltpu.sync_copy(acc.at[pl.ds(base, STRIPE)], out_ref.at[pl.ds(base, STRIPE)])

    @pl.when(s == NS - 1)
    def _():
        pltpu.sync_copy(
            acc.at[pl.ds(base, LAST_STRIPE)], out_ref.at[pl.ds(base, LAST_STRIPE)]
        )


def _degree_body(idx_hbm, out_hbm, idx_v, hist_v, rid_v, acc):
    c = lax.axis_index("c")
    s = lax.axis_index("s")
    ones16 = jnp.full((L,), 1.0, jnp.float32)
    zeros16 = jnp.zeros((L,), jnp.float32)
    iota16 = lax.iota(jnp.int32, L)
    ept = E // NS
    hrows = NP // L

    pltpu.sync_copy(idx_hbm.at[c, pl.ds(s * ept, ept)], idx_v)

    @pl.loop(0, hrows)
    def _(j):
        hist_v[j] = zeros16

    for k in range(hrows // 128):
        for m in range(8):
            rid_v[k, pl.ds(m * L, L)] = iota16 + (k * 128 + m * L)

    pltpu.sync_copy(hist_v.at[pl.ds(s * 40, 40)], acc.at[pl.ds(s * 40, 40)])
    plsc.subcore_barrier()

    @pl.loop(0, ept // L)
    def _(i):
        idx16 = idx_v[pl.ds(i * L, L)]
        row16 = lax.shift_right_logical(idx16, 4)
        col16 = lax.bitwise_and(idx16, 15)
        plsc.addupdate_scatter(hist_v, [row16, col16], ones16)

    for k in range(hrows // 128):
        pltpu.sync_copy(
            hist_v.at[pl.ds(k * 128, 128)], acc.at[rid_v.at[k]], add=True
        )
    plsc.subcore_barrier()

    pltpu.sync_copy(acc.at[pl.ds(s * 40, 40)], out_hbm.at[c, pl.ds(s * 40, 40)])


_degree_call = functools.partial(
    pl.kernel,
    out_type=jax.ShapeDtypeStruct((NC, NP // L, L), jnp.float32),
    mesh=_MESH,
    scratch_types=[
        pltpu.VMEM((E // NS,), jnp.int32),
        pltpu.VMEM((NP // L, L), jnp.float32),
        pltpu.VMEM((NP // L // 128, 128), jnp.int32),
        pltpu.VMEM_SHARED((NP // L, L), jnp.float32),
    ],
    compiler_params=_SC_PARAMS_NL,
)(_degree_body)


NBUF = 6


def _scatter_body(
    hlo_hbm, hhi_hbm, sidx_hbm, ridx_hbm, olo_hbm, ohi_hbm,
    sidx_v, ridx_v, rb0, rb1, rb2, rb3, rb4, rb5, acc,
    gs0, gs1, gs2, gs3, gs4, gs5, isem,
):
    rows_bufs = (rb0, rb1, rb2, rb3, rb4, rb5)
    gsems = (gs0, gs1, gs2, gs3, gs4, gs5)
    c = lax.axis_index("c")
    s = lax.axis_index("s")
    zeros16 = jnp.zeros((L,), jnp.float32)

    pltpu.async_copy(sidx_hbm.at[s], sidx_v, isem)
    pltpu.async_copy(ridx_hbm.at[s], ridx_v, isem)

    @pl.loop(0, CK)
    def _(j):
        for k in range(DH // L):
            rb0[j, pl.ds(k * L, L)] = zeros16

    base = s * STRIPE
    _zero_stripe(rb0, acc, base)
    pltpu.make_async_copy(sidx_hbm.at[s], sidx_v, isem).wait()
    pltpu.make_async_copy(ridx_hbm.at[s], ridx_v, isem).wait()
    plsc.subcore_barrier()

    def gather(j, rows, gsem):
        @pl.when(c == 0)
        def _():
            pltpu.async_copy(hlo_hbm.at[sidx_v.at[j]], rows, gsem)

        @pl.when(c == 1)
        def _():
            pltpu.async_copy(hhi_hbm.at[sidx_v.at[j]], rows, gsem)

    def gather_wait(j, rows, gsem):
        pltpu.make_async_copy(hlo_hbm.at[sidx_v.at[j]], rows, gsem).wait()

    for b in range(NBUF):
        gather(b, rows_bufs[b], gsems[b])

    NFULL = NCH // NBUF
    NREM = NCH - NFULL * NBUF

    @pl.loop(0, NFULL)
    def _(g):
        a = g * NBUF
        for b in range(NBUF):
            j = a + b
            gather_wait(j, rows_bufs[b], gsems[b])
            pltpu.sync_copy(rows_bufs[b], acc.at[ridx_v.at[j]], add=True)
            jn = j + NBUF

            @pl.when(jn < NCH)
            def _():
                gather(jn, rows_bufs[b], gsems[b])

    for b in range(NREM):
        j = NFULL * NBUF + b
        gather_wait(j, rows_bufs[b], gsems[b])
        pltpu.sync_copy(rows_bufs[b], acc.at[ridx_v.at[j]], add=True)

    plsc.subcore_barrier()

    @pl.when(c == 0)
    def _():
        _stripe_out(acc, olo_hbm, s)

    @pl.when(c == 1)
    def _():
        _stripe_out(acc, ohi_hbm, s)


_scatter_call = functools.partial(
    pl.kernel,
    out_type=[
        jax.ShapeDtypeStruct((N, DH), jnp.float32),
        jax.ShapeDtypeStruct((N, DH), jnp.float32),
    ],
    mesh=_MESH,
    scratch_types=[
        pltpu.VMEM((NCH, CK), jnp.int32),
        pltpu.VMEM((NCH, CK), jnp.int32),
        pltpu.VMEM((CK, DH), jnp.float32),
        pltpu.VMEM((CK, DH), jnp.float32),
        pltpu.VMEM((CK, DH), jnp.float32),
        pltpu.VMEM((CK, DH), jnp.float32),
        pltpu.VMEM((CK, DH), jnp.float32),
        pltpu.VMEM((CK, DH), jnp.float32),
        pltpu.VMEM_SHARED((N, DH), jnp.float32),
        pltpu.SemaphoreType.DMA,
        pltpu.SemaphoreType.DMA,
        pltpu.SemaphoreType.DMA,
        pltpu.SemaphoreType.DMA,
        pltpu.SemaphoreType.DMA,
        pltpu.SemaphoreType.DMA,
        pltpu.SemaphoreType.DMA,
    ],
    compiler_params=_SC_PARAMS,
)(_scatter_body)


_BLK = 1000
_TC_PARAMS = pltpu.CompilerParams(dimension_semantics=("parallel",))


def _l1_body(x_ref, w_ref, b_ref, d_ref, olo_ref, ohi_ref):
    h = jnp.tanh(
        jnp.dot(x_ref[...], w_ref[...], preferred_element_type=jnp.float32)
        + b_ref[...]
    )
    dcol = d_ref[...]
    h = h * lax.rsqrt(jnp.maximum(dcol + 1.0, 1.0))
    olo_ref[...] = h[:, :DH]
    ohi_ref[...] = h[:, DH:]


def _l2_body(plo_ref, phi_ref, hlo_ref, hhi_ref, dr_ref, w_ref, b_ref, ds_ref,
             olo_ref, ohi_ref):
    agg = jnp.concatenate(
        [plo_ref[...] + hlo_ref[...], phi_ref[...] + hhi_ref[...]], axis=1
    ) * lax.rsqrt(jnp.maximum(dr_ref[...] + 1.0, 1.0))
    h2 = jnp.dot(agg, w_ref[...], preferred_element_type=jnp.float32) + b_ref[...]
    h2 = h2 * lax.rsqrt(jnp.maximum(ds_ref[...], 1.0))
    olo_ref[...] = h2[:, :DH]
    ohi_ref[...] = h2[:, DH:]


def _fin_body(qlo_ref, qhi_ref, dr_ref, o_ref):
    o_ref[...] = jnp.concatenate(
        [qlo_ref[...], qhi_ref[...]], axis=1
    ) * lax.rsqrt(jnp.maximum(dr_ref[...], 1.0))


def _row_spec(i_dim):
    return pl.BlockSpec((_BLK, i_dim), lambda i: (i, 0))


def _full_spec(r, c):
    return pl.BlockSpec((r, c), lambda i: (0, 0))


def _tc_layer1(x, W1, b1, degs):
    return pl.pallas_call(
        _l1_body,
        grid=(N // _BLK,),
        compiler_params=_TC_PARAMS,
        in_specs=[_row_spec(D), _full_spec(D, D), _full_spec(1, D), _row_spec(1)],
        out_specs=[_row_spec(DH), _row_spec(DH)],
        out_shape=[
            jax.ShapeDtypeStruct((N, DH), jnp.float32),
            jax.ShapeDtypeStruct((N, DH), jnp.float32),
        ],
    )(x, W1, b1.reshape(1, D), degs)


def _tc_layer2(plo, phi, hlo, hhi, degr, W2, b2, degs):
    return pl.pallas_call(
        _l2_body,
        grid=(N // _BLK,),
        compiler_params=_TC_PARAMS,
        in_specs=[
            _row_spec(DH),
            _row_spec(DH),
            _row_spec(DH),
            _row_spec(DH),
            _row_spec(1),
            _full_spec(D, D),
            _full_spec(1, D),
            _row_spec(1),
        ],
        out_specs=[_row_spec(DH), _row_spec(DH)],
        out_shape=[
            jax.ShapeDtypeStruct((N, DH), jnp.float32),
            jax.ShapeDtypeStruct((N, DH), jnp.float32),
        ],
    )(plo, phi, hlo, hhi, degr, W2, b2.reshape(1, D), degs)


def _tc_final(qlo, qhi, degr):
    return pl.pallas_call(
        _fin_body,
        grid=(N // _BLK,),
        compiler_params=_TC_PARAMS,
        in_specs=[_row_spec(DH), _row_spec(DH), _row_spec(1)],
        out_specs=_row_spec(D),
        out_shape=jax.ShapeDtypeStruct((N, D), jnp.float32),
    )(qlo, qhi, degr)


@jax.jit
def kernel(x, senders, receivers, W1, b1, W2, b2):
    senders = senders.astype(jnp.int32)
    receivers = receivers.astype(jnp.int32)

    sidx = senders.reshape(NS, NCH, CK)
    ridx = receivers.reshape(NS, NCH, CK)
    deg_in = jnp.stack([senders, receivers])
    deg = _degree_call(deg_in).reshape(NC, NP)[:, :N]
    degs = deg[0].reshape(N, 1)
    degr = deg[1].reshape(N, 1)

    h1lo, h1hi = _tc_layer1(x, W1, b1, degs)
    plo, phi = _scatter_call(h1lo, h1hi, sidx, ridx)
    h2lo, h2hi = _tc_layer2(plo, phi, h1lo, h1hi, degr, W2, b2, degs)
    qlo, qhi = _scatter_call(h2lo, h2hi, sidx, ridx)
    return _tc_final(qlo, qhi, degr)


# speedup vs baseline: 13.4624x; 1.0090x over previous
import functools

import jax
import jax.numpy as jnp
from jax import lax
from jax.experimental import pallas as pl
from jax.experimental.pallas import tpu as pltpu
from jax.experimental.pallas import tpu_sc as plsc

N = 10000
E = 320000
D = 128
DH = D // 2
NC = 2
NS = 16
L = 16
CK = 125
NCH = (E // NS) // CK
STRIPE = 624
LAST_STRIPE = N - (NS - 1) * STRIPE
NP = 10240

_MESH = plsc.VectorSubcoreMesh(
    core_axis_name="c", subcore_axis_name="s", num_cores=NC, num_subcores=NS
)
_SC_PARAMS = pltpu.CompilerParams(use_tc_tiling_on_sc=False, skip_device_barrier=True)
_SC_PARAMS_NL = pltpu.CompilerParams(
    use_tc_tiling_on_sc=False, skip_device_barrier=True, needs_layout_passes=False
)


def _zero_stripe(zsrc, acc, base):
    for k in range(5):
        pltpu.sync_copy(zsrc.at[pl.ds(0, 120)], acc.at[pl.ds(base + 120 * k, 120)])
    pltpu.sync_copy(zsrc.at[pl.ds(0, 40)], acc.at[pl.ds(base + 600, 40)])


def _stripe_out(acc, out_ref, s):
    base = s * STRIPE

    @pl.when(s < NS - 1)
    def _():
        pltpu.sync_copy(acc.at[pl.ds(base, STRIPE)], out_ref.at[pl.ds(base, STRIPE)])

    @pl.when(s == NS - 1)
    def _():
        pltpu.sync_copy(
            acc.at[pl.ds(base, LAST_STRIPE)], out_ref.at[pl.ds(base, LAST_STRIPE)]
        )


def _degree_body(idx_hbm, out_hbm, idx_v, hist_v, rid_v, acc):
    c = lax.axis_index("c")
    s = lax.axis_index("s")
    ones16 = jnp.full((L,), 1.0, jnp.float32)
    zeros16 = jnp.zeros((L,), jnp.float32)
    iota16 = lax.iota(jnp.int32, L)
    ept = E // NS
    hrows = NP // L

    pltpu.sync_copy(idx_hbm.at[c, pl.ds(s * ept, ept)], idx_v)

    @pl.loop(0, hrows)
    def _(j):
        hist_v[j] = zeros16

    for k in range(hrows // 128):
        for m in range(8):
            rid_v[k, pl.ds(m * L, L)] = iota16 + (k * 128 + m * L)

    pltpu.sync_copy(hist_v.at[pl.ds(s * 40, 40)], acc.at[pl.ds(s * 40, 40)])
    plsc.subcore_barrier()

    @pl.loop(0, ept // L)
    def _(i):
        idx16 = idx_v[pl.ds(i * L, L)]
        row16 = lax.shift_right_logical(idx16, 4)
        col16 = lax.bitwise_and(idx16, 15)
        plsc.addupdate_scatter(hist_v, [row16, col16], ones16)

    for k in range(hrows // 128):
        pltpu.sync_copy(
            hist_v.at[pl.ds(k * 128, 128)], acc.at[rid_v.at[k]], add=True
        )
    plsc.subcore_barrier()

    pltpu.sync_copy(acc.at[pl.ds(s * 40, 40)], out_hbm.at[c, pl.ds(s * 40, 40)])


_degree_call = functools.partial(
    pl.kernel,
    out_type=jax.ShapeDtypeStruct((NC, NP // L, L), jnp.float32),
    mesh=_MESH,
    scratch_types=[
        pltpu.VMEM((E // NS,), jnp.int32),
        pltpu.VMEM((NP // L, L), jnp.float32),
        pltpu.VMEM((NP // L // 128, 128), jnp.int32),
        pltpu.VMEM_SHARED((NP // L, L), jnp.float32),
    ],
    compiler_params=_SC_PARAMS_NL,
)(_degree_body)


NBUF = 6


def _scatter_body(
    hlo_hbm, hhi_hbm, sidx_hbm, ridx_hbm, olo_hbm, ohi_hbm,
    sidx_v, ridx_v, rb0, rb1, rb2, rb3, rb4, rb5, acc,
    gs0, gs1, gs2, gs3, gs4, gs5, isem,
):
    rows_bufs = (rb0, rb1, rb2, rb3, rb4, rb5)
    gsems = (gs0, gs1, gs2, gs3, gs4, gs5)
    c = lax.axis_index("c")
    s = lax.axis_index("s")
    zeros16 = jnp.zeros((L,), jnp.float32)

    pltpu.async_copy(sidx_hbm.at[s], sidx_v, isem)
    pltpu.async_copy(ridx_hbm.at[s], ridx_v, isem)

    @pl.loop(0, CK)
    def _(j):
        for k in range(DH // L):
            rb0[j, pl.ds(k * L, L)] = zeros16

    base = s * STRIPE
    _zero_stripe(rb0, acc, base)
    pltpu.make_async_copy(sidx_hbm.at[s], sidx_v, isem).wait()
    pltpu.make_async_copy(ridx_hbm.at[s], ridx_v, isem).wait()
    plsc.subcore_barrier()

    def gather(j, rows, gsem):
        @pl.when(c == 0)
        def _():
            pltpu.async_copy(hlo_hbm.at[sidx_v.at[j]], rows, gsem)

        @pl.when(c == 1)
        def _():
            pltpu.async_copy(hhi_hbm.at[sidx_v.at[j]], rows, gsem)

    def gather_wait(j, rows, gsem):
        pltpu.make_async_copy(hlo_hbm.at[sidx_v.at[j]], rows, gsem).wait()

    for b in range(NBUF):
        gather(b, rows_bufs[b], gsems[b])

    NFULL = NCH // NBUF
    NREM = NCH - NFULL * NBUF

    @pl.loop(0, NFULL)
    def _(g):
        a = g * NBUF
        for b in range(NBUF):
            j = a + b
            gather_wait(j, rows_bufs[b], gsems[b])
            pltpu.sync_copy(rows_bufs[b], acc.at[ridx_v.at[j]], add=True)
            jn = j + NBUF

            @pl.when(jn < NCH)
            def _():
                gather(jn, rows_bufs[b], gsems[b])

    for b in range(NREM):
        j = NFULL * NBUF + b
        gather_wait(j, rows_bufs[b], gsems[b])
        pltpu.sync_copy(rows_bufs[b], acc.at[ridx_v.at[j]], add=True)

    plsc.subcore_barrier()

    @pl.when(c == 0)
    def _():
        _stripe_out(acc, olo_hbm, s)

    @pl.when(c == 1)
    def _():
        _stripe_out(acc, ohi_hbm, s)


_scatter_call = functools.partial(
    pl.kernel,
    out_type=[
        jax.ShapeDtypeStruct((N, DH), jnp.float32),
        jax.ShapeDtypeStruct((N, DH), jnp.float32),
    ],
    mesh=_MESH,
    scratch_types=[
        pltpu.VMEM((NCH, CK), jnp.int32),
        pltpu.VMEM((NCH, CK), jnp.int32),
        pltpu.VMEM((CK, DH), jnp.float32),
        pltpu.VMEM((CK, DH), jnp.float32),
        pltpu.VMEM((CK, DH), jnp.float32),
        pltpu.VMEM((CK, DH), jnp.float32),
        pltpu.VMEM((CK, DH), jnp.float32),
        pltpu.VMEM((CK, DH), jnp.float32),
        pltpu.VMEM_SHARED((N, DH), jnp.float32),
        pltpu.SemaphoreType.DMA,
        pltpu.SemaphoreType.DMA,
        pltpu.SemaphoreType.DMA,
        pltpu.SemaphoreType.DMA,
        pltpu.SemaphoreType.DMA,
        pltpu.SemaphoreType.DMA,
        pltpu.SemaphoreType.DMA,
    ],
    compiler_params=_SC_PARAMS,
)(_scatter_body)


_BLK = 2000
_TC_PARAMS = pltpu.CompilerParams(dimension_semantics=("parallel",))


def _l1_body(x_ref, w_ref, b_ref, d_ref, olo_ref, ohi_ref):
    h = jnp.tanh(
        jnp.dot(x_ref[...], w_ref[...], preferred_element_type=jnp.float32)
        + b_ref[...]
    )
    dcol = d_ref[...][:, 0:1]
    h = h * lax.rsqrt(jnp.maximum(dcol + 1.0, 1.0))
    olo_ref[...] = h[:, :DH]
    ohi_ref[...] = h[:, DH:]


def _l2_body(plo_ref, phi_ref, hlo_ref, hhi_ref, w_ref, b_ref, d_ref,
             olo_ref, ohi_ref):
    agg = jnp.concatenate(
        [plo_ref[...] + hlo_ref[...], phi_ref[...] + hhi_ref[...]], axis=1
    ) * lax.rsqrt(jnp.maximum(d_ref[...][:, 1:2] + 1.0, 1.0))
    h2 = jnp.dot(agg, w_ref[...], preferred_element_type=jnp.float32) + b_ref[...]
    h2 = h2 * lax.rsqrt(jnp.maximum(d_ref[...][:, 0:1], 1.0))
    olo_ref[...] = h2[:, :DH]
    ohi_ref[...] = h2[:, DH:]


def _fin_body(qlo_ref, qhi_ref, dr_ref, o_ref):
    o_ref[...] = jnp.concatenate(
        [qlo_ref[...], qhi_ref[...]], axis=1
    ) * lax.rsqrt(jnp.maximum(dr_ref[...][:, 1:2], 1.0))


def _row_spec(i_dim):
    return pl.BlockSpec((_BLK, i_dim), lambda i: (i, 0))


def _full_spec(r, c):
    return pl.BlockSpec((r, c), lambda i: (0, 0))


def _tc_layer1(x, W1, b1, degp):
    return pl.pallas_call(
        _l1_body,
        grid=(N // _BLK,),
        compiler_params=_TC_PARAMS,
        in_specs=[_row_spec(D), _full_spec(D, D), _full_spec(1, D), _row_spec(2)],
        out_specs=[_row_spec(DH), _row_spec(DH)],
        out_shape=[
            jax.ShapeDtypeStruct((N, DH), jnp.float32),
            jax.ShapeDtypeStruct((N, DH), jnp.float32),
        ],
    )(x, W1, b1.reshape(1, D), degp)


def _tc_layer2(plo, phi, hlo, hhi, W2, b2, degp):
    return pl.pallas_call(
        _l2_body,
        grid=(N // _BLK,),
        compiler_params=_TC_PARAMS,
        in_specs=[
            _row_spec(DH),
            _row_spec(DH),
            _row_spec(DH),
            _row_spec(DH),
            _full_spec(D, D),
            _full_spec(1, D),
            _row_spec(2),
        ],
        out_specs=[_row_spec(DH), _row_spec(DH)],
        out_shape=[
            jax.ShapeDtypeStruct((N, DH), jnp.float32),
            jax.ShapeDtypeStruct((N, DH), jnp.float32),
        ],
    )(plo, phi, hlo, hhi, W2, b2.reshape(1, D), degp)


def _tc_final(qlo, qhi, degp):
    return pl.pallas_call(
        _fin_body,
        grid=(N // _BLK,),
        compiler_params=_TC_PARAMS,
        in_specs=[_row_spec(DH), _row_spec(DH), _row_spec(2)],
        out_specs=_row_spec(D),
        out_shape=jax.ShapeDtypeStruct((N, D), jnp.float32),
    )(qlo, qhi, degp)


@jax.jit
def kernel(x, senders, receivers, W1, b1, W2, b2):
    senders = senders.astype(jnp.int32)
    receivers = receivers.astype(jnp.int32)

    sidx = senders.reshape(NS, NCH, CK)
    ridx = receivers.reshape(NS, NCH, CK)
    deg_in = jnp.stack([senders, receivers])
    deg = _degree_call(deg_in).reshape(NC, NP)[:, :N]
    degp = deg.T

    h1lo, h1hi = _tc_layer1(x, W1, b1, degp)
    plo, phi = _scatter_call(h1lo, h1hi, sidx, ridx)
    h2lo, h2hi = _tc_layer2(plo, phi, h1lo, h1hi, W2, b2, degp)
    qlo, qhi = _scatter_call(h2lo, h2hi, sidx, ridx)
    return _tc_final(qlo, qhi, degp)
